# baseline (speedup 1.0000x reference)
import functools

import jax
import jax.numpy as jnp
import numpy as np
from jax.experimental import pallas as pl
from jax.experimental.pallas import tpu as pltpu


def _rvq_body(flat_ref, cb_ref, c2_ref, cb1_ref, cb2_ref, cb3_ref,
              q_ref, codes_ref, loss_ref):
    resid = flat_ref[...]
    acc = jnp.zeros_like(resid)
    n_q, bins, _ = cb_ref.shape
    R = resid.shape[0]
    iota = jax.lax.broadcasted_iota(jnp.int32, (R, bins), 1)
    codes_cols = []
    loss_parts = []
    dn = (((1,), (0,)), ((), ()))
    for q in range(n_q):
        cb = cb_ref[q]
        c2 = c2_ref[q]
        dot = jax.lax.dot_general(
            resid, cb, (((1,), (1,)), ((), ())),
            preferred_element_type=jnp.float32)
        rsq = jnp.sum(resid * resid, axis=1, keepdims=True)
        dist = -(rsq - 2.0 * dot + c2)
        m = jnp.max(dist, axis=1, keepdims=True)
        idx = jnp.min(jnp.where(dist == m, iota, bins),
                      axis=1, keepdims=True)
        onehot = (iota == idx).astype(jnp.bfloat16)
        quant = (jax.lax.dot_general(onehot, cb1_ref[q], dn,
                                     preferred_element_type=jnp.float32)
                 + jax.lax.dot_general(onehot, cb2_ref[q], dn,
                                       preferred_element_type=jnp.float32)
                 + jax.lax.dot_general(onehot, cb3_ref[q], dn,
                                       preferred_element_type=jnp.float32))
        diff = quant - resid
        loss_parts.append(jnp.sum(diff * diff, axis=0, keepdims=True))
        qst = resid + diff
        resid = resid - qst
        acc = acc + qst
        codes_cols.append(idx)
    q_ref[...] = acc
    codes_ref[...] = jnp.concatenate(codes_cols, axis=1)
    loss_ref[0] = jnp.concatenate(loss_parts, axis=0)


def kernel(x, codebooks, sample_rate):
    n_q, bins, D = codebooks.shape
    B, Dx, T = x.shape
    rows = B * T
    CHUNK = 1024
    grid = rows // CHUNK

    flat = x.transpose(0, 2, 1).reshape(rows, D)
    c2 = jnp.sum(codebooks ** 2, axis=-1).reshape(n_q, 1, bins)
    def _trunc(v):
        bits = jax.lax.bitcast_convert_type(v, jnp.uint32)
        return jax.lax.bitcast_convert_type(bits & jnp.uint32(0xFFFF0000),
                                            jnp.float32)
    h1 = _trunc(codebooks)
    r1 = codebooks - h1
    h2 = _trunc(r1)
    cb1 = h1.astype(jnp.bfloat16)
    cb2 = h2.astype(jnp.bfloat16)
    cb3 = (r1 - h2).astype(jnp.bfloat16)

    qrows, codes_rows, loss_parts = pl.pallas_call(
        _rvq_body,
        grid=(grid,),
        in_specs=[
            pl.BlockSpec((CHUNK, D), lambda i: (i, 0)),
            pl.BlockSpec((n_q, bins, D), lambda i: (0, 0, 0)),
            pl.BlockSpec((n_q, 1, bins), lambda i: (0, 0, 0)),
            pl.BlockSpec((n_q, bins, D), lambda i: (0, 0, 0)),
            pl.BlockSpec((n_q, bins, D), lambda i: (0, 0, 0)),
            pl.BlockSpec((n_q, bins, D), lambda i: (0, 0, 0)),
        ],
        out_specs=[
            pl.BlockSpec((CHUNK, D), lambda i: (i, 0)),
            pl.BlockSpec((CHUNK, n_q), lambda i: (i, 0)),
            pl.BlockSpec((1, n_q, D), lambda i: (i, 0, 0)),
        ],
        out_shape=[
            jax.ShapeDtypeStruct((rows, D), jnp.float32),
            jax.ShapeDtypeStruct((rows, n_q), jnp.int32),
            jax.ShapeDtypeStruct((grid, n_q, D), jnp.float32),
        ],
    )(flat, codebooks, c2, cb1, cb2, cb3)

    quantized_out = qrows.reshape(B, T, D).transpose(0, 2, 1)
    codes = codes_rows.reshape(B, T, n_q).transpose(2, 0, 1)
    losses = loss_parts.sum(axis=(0, 2)) / jnp.float32(rows * D)
    commit_loss = jnp.mean(losses)
    bw_per_q = float(np.log2(bins)) * sample_rate / 1000.0
    bw = jnp.asarray(n_q * bw_per_q, dtype=x.dtype)
    return (quantized_out, codes, bw, commit_loss)


# speedup vs baseline: 1.4686x; 1.4686x over previous
import jax
import jax.numpy as jnp
import numpy as np
from jax.experimental import pallas as pl
from jax.experimental.pallas import tpu as pltpu

_NT = (((1,), (1,)), ((), ()))
_NN = (((1,), (0,)), ((), ()))


def _rvq_body(flat_ref, cbs2_ref, c2_ref, cb1_ref, cb2_ref, cb3_ref,
              q_ref, codes_ref, loss_ref):
    n_q, bins, _ = cbs2_ref.shape
    R = flat_ref.shape[0]
    H = R // 2
    iota = jax.lax.broadcasted_iota(jnp.int32, (H, bins), 1)

    resid = [flat_ref[:H, :], flat_ref[H:, :]]
    acc = [jnp.zeros_like(resid[0]) for _ in range(2)]
    codes_cols = [[], []]
    loss_parts = []
    for q in range(n_q):
        cbs2 = cbs2_ref[q]
        c2 = c2_ref[q]
        stage_loss = []
        for h in range(2):
            r = resid[h]
            dot2 = jax.lax.dot_general(
                r, cbs2, _NT, preferred_element_type=jnp.float32)
            rsq = jnp.sum(r * r, axis=1, keepdims=True)
            neg = rsq - dot2 + c2
            m = jnp.min(neg, axis=1, keepdims=True)
            idx = jnp.min(jnp.where(neg == m, iota, bins),
                          axis=1, keepdims=True)
            onehot = (iota == idx).astype(jnp.bfloat16)
            quant = (jax.lax.dot_general(onehot, cb1_ref[q], _NN,
                                         preferred_element_type=jnp.float32)
                     + jax.lax.dot_general(onehot, cb2_ref[q], _NN,
                                           preferred_element_type=jnp.float32)
                     + jax.lax.dot_general(onehot, cb3_ref[q], _NN,
                                           preferred_element_type=jnp.float32))
            diff = quant - r
            stage_loss.append(jnp.sum(diff * diff, axis=0, keepdims=True))
            qst = r + diff
            resid[h] = r - qst
            acc[h] = acc[h] + qst
            codes_cols[h].append(idx)
        loss_parts.append(stage_loss[0] + stage_loss[1])
    q_ref[:H, :] = acc[0]
    q_ref[H:, :] = acc[1]
    codes_ref[:H, :] = jnp.concatenate(codes_cols[0], axis=1)
    codes_ref[H:, :] = jnp.concatenate(codes_cols[1], axis=1)
    loss_ref[0] = jnp.concatenate(loss_parts, axis=0)


def kernel(x, codebooks, sample_rate):
    n_q, bins, D = codebooks.shape
    B, Dx, T = x.shape
    rows = B * T
    CHUNK = 1024
    grid = rows // CHUNK

    flat = x.transpose(0, 2, 1).reshape(rows, D)
    c2 = jnp.sum(codebooks ** 2, axis=-1).reshape(n_q, 1, bins)
    cbs2 = codebooks * 2.0
    def _trunc(v):
        bits = jax.lax.bitcast_convert_type(v, jnp.uint32)
        return jax.lax.bitcast_convert_type(bits & jnp.uint32(0xFFFF0000),
                                            jnp.float32)
    h1 = _trunc(codebooks)
    r1 = codebooks - h1
    h2 = _trunc(r1)
    cb1 = h1.astype(jnp.bfloat16)
    cb2 = h2.astype(jnp.bfloat16)
    cb3 = (r1 - h2).astype(jnp.bfloat16)

    qrows, codes_rows, loss_parts = pl.pallas_call(
        _rvq_body,
        grid=(grid,),
        in_specs=[
            pl.BlockSpec((CHUNK, D), lambda i: (i, 0)),
            pl.BlockSpec((n_q, bins, D), lambda i: (0, 0, 0)),
            pl.BlockSpec((n_q, 1, bins), lambda i: (0, 0, 0)),
            pl.BlockSpec((n_q, bins, D), lambda i: (0, 0, 0)),
            pl.BlockSpec((n_q, bins, D), lambda i: (0, 0, 0)),
            pl.BlockSpec((n_q, bins, D), lambda i: (0, 0, 0)),
        ],
        out_specs=[
            pl.BlockSpec((CHUNK, D), lambda i: (i, 0)),
            pl.BlockSpec((CHUNK, n_q), lambda i: (i, 0)),
            pl.BlockSpec((1, n_q, D), lambda i: (i, 0, 0)),
        ],
        out_shape=[
            jax.ShapeDtypeStruct((rows, D), jnp.float32),
            jax.ShapeDtypeStruct((rows, n_q), jnp.int32),
            jax.ShapeDtypeStruct((grid, n_q, D), jnp.float32),
        ],
    )(flat, cbs2, c2, cb1, cb2, cb3)

    quantized_out = qrows.reshape(B, T, D).transpose(0, 2, 1)
    codes = codes_rows.reshape(B, T, n_q).transpose(2, 0, 1)
    losses = loss_parts.sum(axis=(0, 2)) / jnp.float32(rows * D)
    commit_loss = jnp.mean(losses)
    bw_per_q = float(np.log2(bins)) * sample_rate / 1000.0
    bw = jnp.asarray(n_q * bw_per_q, dtype=x.dtype)
    return (quantized_out, codes, bw, commit_loss)
